# baseline (speedup 1.0000x reference)
import functools

import jax
import jax.numpy as jnp
import numpy as np
from jax import lax
from jax.experimental import pallas as pl
from jax.experimental.pallas import tpu as pltpu
from jax.experimental.pallas import tpu_sc as plsc

T, C, H, W = 3, 64, 128, 128
PS = 7
STRIDE0 = 4
K = 7
K_S = 50

NH, NW_ = H // STRIDE0, W // STRIDE0
NQ = T * NH * NW_
H0P, W0P = H + 3, W + 3
H1P, W1P = H + 6, W + 6
W0P2 = W0P + 1
U0 = W0P2 // 2
N0 = T * H0P * U0
N1 = T * H1P * W1P
RW = 2 * C

NWORK = 32
QPW = NQ // NWORK
CHUNK = 9
NCHUNK = 6
UPC = 4 * PS
CROWS = CHUNK * UPC
CROWS_PAD = 256
QUNITS = UPC
QUNITS_PAD = 32
NEG = -3.0e38


def _consts():
    e = np.minimum(np.arange(CROWS_PAD), CROWS - 1)
    u1 = e % UPC
    off_e = ((u1 // 4) * W1P + 2 * (u1 % 4)).astype(np.int32)
    u = np.arange(128)
    off_q = np.where(u < QUNITS, (u // 4) * U0 + (u % 4), 0).astype(np.int32)
    return (jnp.asarray(off_e), jnp.asarray(off_q))


def _sc_search(v0r, v1r, inds96, offe, offq):
    mesh = plsc.VectorSubcoreMesh(core_axis_name="c", subcore_axis_name="s")

    @functools.partial(
        pl.kernel,
        out_type=(
            jax.ShapeDtypeStruct((NQ * 16,), jnp.float32),
            jax.ShapeDtypeStruct((NQ * 16,), jnp.int32),
        ),
        mesh=mesh,
        compiler_params=pltpu.CompilerParams(use_tc_tiling_on_sc=False),
        scratch_types=[
            pltpu.VMEM((CROWS_PAD,), jnp.int32),
            pltpu.VMEM((128,), jnp.int32),
            pltpu.VMEM((2, 128), jnp.int32),
            pltpu.VMEM((2, 128), jnp.int32),
            pltpu.VMEM((2, 128), jnp.int32),
            pltpu.VMEM((2, 2, 128), jnp.int32),
            pltpu.VMEM((2, QUNITS_PAD, RW), jnp.float32),
            pltpu.VMEM((2, CROWS_PAD, RW), jnp.float32),
            pltpu.VMEM((128,), jnp.float32),
            pltpu.VMEM((QPW * 16,), jnp.float32),
            pltpu.VMEM((QPW * 16,), jnp.int32),
            pltpu.SemaphoreType.DMA,
            pltpu.SemaphoreType.DMA,
            pltpu.SemaphoreType.DMA,
            pltpu.SemaphoreType.DMA,
            pltpu.SemaphoreType.DMA,
            pltpu.SemaphoreType.DMA,
        ],
    )
    def body(v0_h, v1_h, inds_h, offe_h, offq_h,
             topd_o, topi_o,
             offe_v, offq_v, indsrow_v, qidx_v, cbase_v, cidx_v,
             qbuf, pbuf, dists_v, topd_l, topi_l,
             semi0, semi1, semq0, semq1, semg0, semg1):
        wid = lax.axis_index("s") * 2 + lax.axis_index("c")
        semi = (semi0, semi1)
        semq = (semq0, semq1)
        semg = (semg0, semg1)
        iota = lax.iota(jnp.int32, 16)

        pltpu.sync_copy(offe_h, offe_v)
        pltpu.sync_copy(offq_h, offq_v)

        def _red(x, sop):
            y = sop(x, lax.rev(x, (0,)))
            s = y[0]
            for i in range(1, 8):
                s = sop(s, y[i])
            return s

        def decode_cbase(par):
            for v in range(NCHUNK):
                iv = indsrow_v[par, pl.ds(v * 16, 16)]
                t1 = iv >> 14
                h1 = (iv >> 7) & 127
                w1 = iv & 127
                cbase_v[par, pl.ds(v * 16, 16)] = (t1 * H1P + h1) * W1P + w1

        def inds_copy(q, par):
            return pltpu.make_async_copy(
                inds_h.at[q], indsrow_v.at[par], semi[par])

        def qbuf_copy(par):
            return pltpu.make_async_copy(
                v0_h.at[qidx_v.at[par, pl.ds(0, QUNITS_PAD)]],
                qbuf.at[par], semq[par])

        def fire_qpatch(q, par):
            qt = q >> 10
            rem = q & 1023
            qh = (rem >> 5) * STRIDE0
            qw = (rem & 31) * STRIDE0
            qbase = (qt * H0P + qh) * U0 + (qw >> 1)
            for v in range(QUNITS_PAD // 16):
                qidx_v[par, pl.ds(v * 16, 16)] = (
                    offq_v[pl.ds(v * 16, 16)] + qbase)
            qbuf_copy(par).start()

        def chunk_copies(slot):
            return [
                pltpu.make_async_copy(
                    v1_h.at[cidx_v.at[slot, j]],
                    pbuf.at[slot, pl.ds(j * 128, 128)],
                    semg[slot])
                for j in range(2)
            ]

        def fire_chunk(slot, cin, par):
            sel = cin < NCHUNK
            par_idx = jnp.where(sel, par, 1 - par)
            row = jnp.where(sel, cin, 0)
            cbrow = cbase_v[par_idx, pl.ds(row * 16, 16)]
            for i in range(CROWS_PAD // 16):
                e0 = 16 * i
                l0 = min(e0, CROWS - 1) // UPC
                l1 = min(e0 + 15, CROWS - 1) // UPC
                oe = offe_v[pl.ds(e0, 16)]
                cb0 = cbrow[l0]
                if l1 == l0:
                    idx = oe + cb0
                else:
                    split = l1 * UPC - e0
                    idx = oe + jnp.where(iota < split, cb0, cbrow[l1])
                cidx_v[slot, i // 8, pl.ds((i % 8) * 16, 16)] = idx
            for cp in chunk_copies(slot):
                cp.start()

        def compute_chunk(slot, ci, par):
            zero = jnp.zeros((16,), jnp.float32)

            def pbody(di, accs):
                out = list(accs)
                for dj in range(PS):
                    for c4 in range(C // 16):
                        col = (dj & 1) * C + c4 * 16
                        ur = di * 4 + dj // 2
                        qv = qbuf[par, ur, pl.ds(col, 16)]
                        for l in range(CHUNK):
                            cv = pbuf[slot, l * UPC + ur, pl.ds(col, 16)]
                            out[l] = out[l] + qv * cv
                return tuple(out)

            accs = lax.fori_loop(0, PS, pbody, (zero,) * CHUNK)
            dvec = jnp.full((16,), NEG, jnp.float32)
            for l in range(CHUNK):
                dvec = jnp.where(iota == l, _red(accs[l], jnp.add), dvec)
            dists_v[pl.ds(ci * 16, 16)] = dvec

        def proc_query(qi, par):
            q = wid * QPW + qi
            qn1 = jnp.minimum(q + 1, NQ - 1)
            qn2 = jnp.minimum(q + 2, NQ - 1)
            ii = [indsrow_v[par, pl.ds(ci * 16, 16)] for ci in range(NCHUNK)]
            inds_copy(qn2, par).start()
            inds_copy(qn1, 1 - par).wait()
            decode_cbase(1 - par)
            qbuf_copy(par).wait()
            fire_qpatch(qn1, 1 - par)
            qt = q >> 10
            rem = q & 1023
            qh = (rem >> 5) * STRIDE0
            qw = (rem & 31) * STRIDE0
            qflat = qt * (H * W) + qh * W + qw

            def chunk_pair(cpi, carry):
                c0 = 2 * cpi
                fire_chunk(1, c0 + 1, par)
                for cp in chunk_copies(0):
                    cp.wait()
                compute_chunk(0, c0, par)
                fire_chunk(0, c0 + 2, par)
                for cp in chunk_copies(1):
                    cp.wait()
                compute_chunk(1, c0 + 1, par)
                return carry

            lax.fori_loop(0, NCHUNK // 2, chunk_pair, 0)

            d = []
            for ci in range(NCHUNK):
                dv = dists_v[pl.ds(ci * 16, 16)]
                if ci == NCHUNK - 1:
                    dv = jnp.where(iota >= K_S - CHUNK * (NCHUNK - 1),
                                   NEG, dv)
                d.append(jnp.where(ii[ci] == qflat, dv + 1.0e4, dv))
            outd = jnp.zeros((16,), jnp.float32)
            outi = jnp.zeros((16,), jnp.int32)
            for j in range(K):
                mv = d[0]
                for ci in range(1, NCHUNK):
                    mv = jnp.maximum(mv, d[ci])
                m = _red(mv, jnp.maximum)
                pos = None
                for ci in range(NCHUNK):
                    pk = jnp.where(d[ci] == m, iota + ci * 16,
                                   jnp.int32(9999))
                    pos = pk if pos is None else jnp.minimum(pos, pk)
                pos = _red(pos, jnp.minimum)
                isel = jnp.zeros((16,), jnp.int32)
                for ci in range(NCHUNK):
                    selk = (iota + ci * 16) == pos
                    isel = isel + jnp.where(selk, ii[ci], 0)
                    d[ci] = jnp.where(selk, NEG, d[ci])
                ival = _red(isel, jnp.add)
                outd = jnp.where(iota == j, m, outd)
                outi = jnp.where(iota == j, ival, outi)
            topd_l[pl.ds(qi * 16, 16)] = outd
            topi_l[pl.ds(qi * 16, 16)] = outi

        q0 = wid * QPW
        pltpu.sync_copy(inds_h.at[q0], indsrow_v.at[0])
        inds_copy(q0 + 1, 1).start()
        decode_cbase(0)
        fire_qpatch(q0, 0)
        fire_chunk(0, jnp.int32(0), 0)

        def pair_body(i, carry):
            proc_query(2 * i, 0)
            proc_query(2 * i + 1, 1)
            return carry

        lax.fori_loop(0, QPW // 2, pair_body, 0)

        inds_copy(q0, 1).wait()
        qbuf_copy(0).wait()
        for cp in chunk_copies(0):
            cp.wait()

        pltpu.sync_copy(topd_l, topd_o.at[pl.ds(wid * (QPW * 16), QPW * 16)])
        pltpu.sync_copy(topi_l, topi_o.at[pl.ds(wid * (QPW * 16), QPW * 16)])

    return body(v0r, v1r, inds96, offe, offq)


def _pair_table(flat):
    n = flat.shape[0]
    ext = jnp.concatenate([flat, flat[-1:]], axis=0)
    return jnp.concatenate([ext[:-1], ext[1:]], axis=1)


def kernel(vid0, vid1, inds_p):
    v0 = jnp.transpose(vid0[0], (0, 2, 3, 1))
    v1 = jnp.transpose(vid1[0], (0, 2, 3, 1))
    v0h = jnp.concatenate(
        [v0[:, 1:2], v0, v0[:, 126:127], v0[:, 125:126]], axis=1)
    v0p = jnp.concatenate(
        [v0h[:, :, 1:2], v0h, v0h[:, :, 126:127], v0h[:, :, 125:126],
         v0h[:, :, 0:1]], axis=2)
    v1h = jnp.concatenate(
        [v1[:, 3:4], v1[:, 2:3], v1[:, 1:2], v1,
         v1[:, 126:127], v1[:, 125:126], v1[:, 124:125]], axis=1)
    v1p = jnp.concatenate(
        [v1h[:, :, 3:4], v1h[:, :, 2:3], v1h[:, :, 1:2], v1h,
         v1h[:, :, 126:127], v1h[:, :, 125:126], v1h[:, :, 124:125]],
        axis=2)
    v0r = v0p.reshape(N0, RW)
    v1r = _pair_table(v1p.reshape(N1, C))

    inds = inds_p[0].astype(jnp.int32)
    inds54 = jnp.pad(inds, ((0, 0), (0, NCHUNK * CHUNK - K_S)))
    inds96 = jnp.pad(
        inds54.reshape(NQ, NCHUNK, CHUNK), ((0, 0), (0, 0), (0, 16 - CHUNK))
    ).reshape(NQ, NCHUNK * 16)
    inds128 = jnp.pad(inds96, ((0, 0), (0, 128 - NCHUNK * 16)))
    offe, offq = _consts()

    topd_flat, topi_flat = _sc_search(v0r, v1r, inds128, offe, offq)

    topd = topd_flat.reshape(NQ, 16)[:, :K][None]
    fi = topi_flat.reshape(NQ, 16)[:, :K]
    it = fi // (H * W)
    ih = (fi // W) % H
    iw = fi % W
    inds_out = jnp.stack([it, ih, iw], axis=-1)[None]
    return topd, inds_out


# speedup vs baseline: 1.9481x; 1.9481x over previous
import functools

import jax
import jax.numpy as jnp
import numpy as np
from jax import lax
from jax.experimental import pallas as pl
from jax.experimental.pallas import tpu as pltpu
from jax.experimental.pallas import tpu_sc as plsc

T, C, H, W = 3, 64, 128, 128
PS = 7
STRIDE0 = 4
K = 7
K_S = 50

NH, NW_ = H // STRIDE0, W // STRIDE0
NQ = T * NH * NW_
H0P, W0P = H + 3, W + 3
H1P, W1P = H + 6, W + 6

NWORK = 32
QPW = NQ // NWORK
CHUNK = 10
NCHUNK = K_S // CHUNK
CROWS = CHUNK * PS * PS
CROWS_PAD = 512
NGATH = CROWS_PAD // 128
QROWS = PS * PS
QROWS_PAD = 56
NEG = -3.0e38


def _consts():
    p = np.arange(PS * PS)
    off1 = (p // PS) * W1P + (p % PS)
    off0 = (p // PS) * W0P + (p % PS)
    e = np.arange(CROWS_PAD)
    off_e = np.where(
        e < CROWS, off1[np.minimum(e, CROWS - 1) % (PS * PS)], 0
    ).astype(np.int32)
    off_q = np.zeros(64, np.int32)
    off_q[:PS * PS] = off0
    return (jnp.asarray(off_e), jnp.asarray(off_q))


def _sc_search(v0r, v1r, inds64, offe, offq):
    mesh = plsc.VectorSubcoreMesh(core_axis_name="c", subcore_axis_name="s")

    @functools.partial(
        pl.kernel,
        out_type=(
            jax.ShapeDtypeStruct((NQ, 16), jnp.float32),
            jax.ShapeDtypeStruct((NQ, 16), jnp.int32),
        ),
        mesh=mesh,
        compiler_params=pltpu.CompilerParams(use_tc_tiling_on_sc=False),
        scratch_types=[
            pltpu.VMEM((CROWS_PAD,), jnp.int32),
            pltpu.VMEM((64,), jnp.int32),
            pltpu.VMEM((2, 64), jnp.int32),
            pltpu.VMEM((2, 64), jnp.int32),
            pltpu.VMEM((2, QROWS_PAD, C), jnp.float32),
            pltpu.VMEM((2, NGATH, 128), jnp.int32),
            pltpu.VMEM((2, CROWS_PAD, C), jnp.float32),
            pltpu.VMEM((QPW, 16), jnp.float32),
            pltpu.VMEM((QPW, 16), jnp.int32),
            pltpu.SemaphoreType.DMA,
            pltpu.SemaphoreType.DMA,
            pltpu.SemaphoreType.DMA,
            pltpu.SemaphoreType.DMA,
            pltpu.SemaphoreType.DMA,
            pltpu.SemaphoreType.DMA,
        ],
    )
    def body(v0_h, v1_h, inds_h, offe_h, offq_h,
             topd_o, topi_o,
             offe_v, offq_v, indsrow_v, qidx_v,
             qbuf, cidx_v, pbuf, topd_l, topi_l,
             semi0, semi1, semq0, semq1, semg0, semg1):
        wid = lax.axis_index("s") * 2 + lax.axis_index("c")
        semi = (semi0, semi1)
        semq = (semq0, semq1)
        semg = (semg0, semg1)
        iota = lax.iota(jnp.int32, 16)

        pltpu.sync_copy(offe_h, offe_v)
        pltpu.sync_copy(offq_h, offq_v)

        def _red(x, sop):
            y = sop(x, lax.rev(x, (0,)))
            s = y[0]
            for i in range(1, 8):
                s = sop(s, y[i])
            return s

        def decode_cbv(par):
            cbv = []
            for v in range(4):
                iv = indsrow_v[par, pl.ds(v * 16, 16)]
                t1 = iv >> 14
                h1 = (iv >> 7) & 127
                w1 = iv & 127
                cbv.append((t1 * H1P + h1) * W1P + w1)
            return cbv

        def inds_copy(q, par):
            return pltpu.make_async_copy(
                inds_h.at[q], indsrow_v.at[par], semi[par])

        def qbuf_copy(par):
            return pltpu.make_async_copy(
                v0_h.at[qidx_v.at[par, pl.ds(0, QROWS_PAD)]],
                qbuf.at[par], semq[par])

        def fire_qpatch(q, par):
            qt = q >> 10
            rem = q & 1023
            qh = (rem >> 5) * STRIDE0
            qw = (rem & 31) * STRIDE0
            qbase = (qt * H0P + qh) * W0P + qw
            for v in range(4):
                qidx_v[par, pl.ds(v * 16, 16)] = (
                    offq_v[pl.ds(v * 16, 16)] + qbase)
            qbuf_copy(par).start()

        def chunk_copies(slot):
            return [
                pltpu.make_async_copy(
                    v1_h.at[cidx_v.at[slot, j]],
                    pbuf.at[slot, pl.ds(j * 128, 128)],
                    semg[slot],
                )
                for j in range(NGATH)
            ]

        def fire_chunk(slot, chunk, cbv):
            for i in range(CROWS_PAD // 16):
                e0 = 16 * i
                l0 = min(e0, CROWS - 1) // (PS * PS)
                l1 = min(e0 + 15, CROWS - 1) // (PS * PS)
                oe = offe_v[pl.ds(e0, 16)]
                a0 = chunk * CHUNK + l0
                cb0 = cbv[a0 // 16][a0 % 16]
                if l1 == l0:
                    idx = oe + cb0
                else:
                    a1 = chunk * CHUNK + l1
                    cb1 = cbv[a1 // 16][a1 % 16]
                    split = l1 * (PS * PS) - e0
                    idx = oe + jnp.where(iota < split, cb0, cb1)
                j, r = divmod(i, 8)
                cidx_v[slot, j, pl.ds(r * 16, 16)] = idx
            for cp in chunk_copies(slot):
                cp.start()

        def compute_chunk(slot, chunk, d, par):
            zero = jnp.zeros((16,), jnp.float32)

            def pbody(p, accs):
                out = list(accs)
                for c4 in range(C // 16):
                    qv = qbuf[par, p, pl.ds(c4 * 16, 16)]
                    for l in range(CHUNK):
                        cv = pbuf[slot, l * QROWS + p, pl.ds(c4 * 16, 16)]
                        out[l] = out[l] + qv * cv
                return tuple(out)

            accs = lax.fori_loop(0, QROWS, pbody, (zero,) * CHUNK)
            for l in range(CHUNK):
                lane = chunk * CHUNK + l
                k, r = divmod(lane, 16)
                s = _red(accs[l], jnp.add)
                d[k] = jnp.where(iota == r, s, d[k])
            return d

        def proc_query(qi, par, cbv_cur):
            q = wid * QPW + qi
            qn = jnp.minimum(q + 1, NQ - 1)
            inds_copy(qn, 1 - par).start()
            fire_qpatch(qn, 1 - par)
            qt = q >> 10
            rem = q & 1023
            qh = (rem >> 5) * STRIDE0
            qw = (rem & 31) * STRIDE0
            qflat = qt * (H * W) + qh * W + qw

            d = [jnp.full((16,), NEG, jnp.float32) for _ in range(4)]
            cbv_next = None
            for ch in range(NCHUNK):
                slot = (ch + par) & 1
                if ch + 1 < NCHUNK:
                    fire_chunk(slot ^ 1, ch + 1, cbv_cur)
                else:
                    inds_copy(qn, 1 - par).wait()
                    cbv_next = decode_cbv(1 - par)
                    fire_chunk(slot ^ 1, 0, cbv_next)
                for cp in chunk_copies(slot):
                    cp.wait()
                if ch == 0:
                    qbuf_copy(par).wait()
                d = compute_chunk(slot, ch, d, par)

            ii = [indsrow_v[par, pl.ds(k * 16, 16)] for k in range(4)]
            for k in range(4):
                d[k] = jnp.where(ii[k] == qflat, d[k] + 1.0e4, d[k])
            outd = jnp.zeros((16,), jnp.float32)
            outi = jnp.zeros((16,), jnp.int32)
            for j in range(K):
                m = _red(jnp.maximum(jnp.maximum(d[0], d[1]),
                                     jnp.maximum(d[2], d[3])), jnp.maximum)
                pos = None
                for k in range(4):
                    pk = jnp.where(d[k] == m, iota + k * 16, jnp.int32(9999))
                    pos = pk if pos is None else jnp.minimum(pos, pk)
                pos = _red(pos, jnp.minimum)
                isel = jnp.zeros((16,), jnp.int32)
                for k in range(4):
                    selk = (iota + k * 16) == pos
                    isel = isel + jnp.where(selk, ii[k], 0)
                    d[k] = jnp.where(selk, NEG, d[k])
                ival = _red(isel, jnp.add)
                outd = jnp.where(iota == j, m, outd)
                outi = jnp.where(iota == j, ival, outi)
            topd_l[qi] = outd
            topi_l[qi] = outi
            return cbv_next

        q0 = wid * QPW
        pltpu.sync_copy(inds_h.at[q0], indsrow_v.at[0])
        fire_qpatch(q0, 0)
        cbv0 = decode_cbv(0)
        fire_chunk(0, 0, cbv0)

        def pair_body(i, cbv):
            cbv = list(cbv)
            cbv = proc_query(2 * i, 0, cbv)
            cbv = proc_query(2 * i + 1, 1, cbv)
            return tuple(cbv)

        lax.fori_loop(0, QPW // 2, pair_body, tuple(cbv0))

        qbuf_copy(0).wait()
        for cp in chunk_copies(0):
            cp.wait()

        pltpu.sync_copy(topd_l, topd_o.at[pl.ds(wid * QPW, QPW)])
        pltpu.sync_copy(topi_l, topi_o.at[pl.ds(wid * QPW, QPW)])

    return body(v0r, v1r, inds64, offe, offq)


def kernel(vid0, vid1, inds_p):
    v0 = jnp.transpose(vid0[0], (0, 2, 3, 1))
    v1 = jnp.transpose(vid1[0], (0, 2, 3, 1))
    v0h = jnp.concatenate(
        [v0[:, 1:2], v0, v0[:, 126:127], v0[:, 125:126]], axis=1)
    v0p = jnp.concatenate(
        [v0h[:, :, 1:2], v0h, v0h[:, :, 126:127], v0h[:, :, 125:126]],
        axis=2)
    v1h = jnp.concatenate(
        [v1[:, 3:4], v1[:, 2:3], v1[:, 1:2], v1,
         v1[:, 126:127], v1[:, 125:126], v1[:, 124:125]], axis=1)
    v1p = jnp.concatenate(
        [v1h[:, :, 3:4], v1h[:, :, 2:3], v1h[:, :, 1:2], v1h,
         v1h[:, :, 126:127], v1h[:, :, 125:126], v1h[:, :, 124:125]],
        axis=2)
    v0r = v0p.reshape(T * H0P * W0P, C)
    v1r = v1p.reshape(T * H1P * W1P, C)
    inds = inds_p[0].astype(jnp.int32)
    inds64 = jnp.pad(inds, ((0, 0), (0, 64 - K_S)))
    offe, offq = _consts()

    topd_pad, topi_pad = _sc_search(v0r, v1r, inds64, offe, offq)

    topd = topd_pad[:, :K][None]
    fi = topi_pad[:, :K]
    it = fi // (H * W)
    ih = (fi // W) % H
    iw = fi % W
    inds_out = jnp.stack([it, ih, iw], axis=-1)[None]
    return topd, inds_out
